# Initial kernel scaffold; baseline (speedup 1.0000x reference)
import jax
import jax.numpy as jnp
from jax.experimental import pallas as pl


def kernel(input, hash):
    raise NotImplementedError("write your pallas kernel here")


# speedup vs baseline: 1.0294x; 1.0294x over previous
import functools

import jax
import jax.numpy as jnp
from jax import lax
from jax.experimental import pallas as pl
from jax.experimental.pallas import tpu as pltpu
from jax.experimental.pallas import tpu_sc as plsc

_info = plsc.get_sparse_core_info()
_NC, _NS = _info.num_cores, _info.num_subcores
_NW = _NC * _NS

_CHUNK = 128


def _make_router(n_tokens, vocab):
    assert n_tokens % (_NW * _CHUNK) == 0
    rows_per_w = n_tokens // (_NW * _CHUNK)
    mesh = plsc.VectorSubcoreMesh(core_axis_name="c", subcore_axis_name="s")

    @functools.partial(
        pl.kernel,
        mesh=mesh,
        out_type=jax.ShapeDtypeStruct((n_tokens // _CHUNK, _CHUNK), jnp.int32),
        scratch_types=[
            pltpu.VMEM((rows_per_w, _CHUNK), jnp.int32),
            pltpu.VMEM((rows_per_w, _CHUNK), jnp.int32),
            pltpu.SemaphoreType.DMA,
        ],
    )
    def router(ids_hbm, table_hbm, out_hbm, idx_v, vals_v, sem):
        wid = lax.axis_index("s") * _NC + lax.axis_index("c")
        row0 = wid * rows_per_w
        pltpu.sync_copy(ids_hbm.at[pl.ds(row0, rows_per_w)], idx_v)
        copies = [
            pltpu.async_copy(table_hbm.at[idx_v.at[j]], vals_v.at[j], sem)
            for j in range(rows_per_w)
        ]
        for c in copies:
            c.wait()
        pltpu.sync_copy(vals_v, out_hbm.at[pl.ds(row0, rows_per_w)])

    return router


def kernel(input, hash):
    b, s = input.shape
    n = b * s
    ids = input.astype(jnp.int32).reshape(n // _CHUNK, _CHUNK)
    out = _make_router(n, hash.shape[0])(ids, hash.astype(jnp.int32))
    return out.reshape(b, s).astype(hash.dtype)


# speedup vs baseline: 1.0315x; 1.0020x over previous
import functools

import jax
import jax.numpy as jnp
from jax import lax
from jax.experimental import pallas as pl
from jax.experimental.pallas import tpu as pltpu
from jax.experimental.pallas import tpu_sc as plsc

_info = plsc.get_sparse_core_info()
_NC, _NS = _info.num_cores, _info.num_subcores
_NW = _NC * _NS


def _make_router(n_tokens):
    assert n_tokens % (8 * _NW) == 0
    per_w = n_tokens // _NW
    mesh = plsc.VectorSubcoreMesh(core_axis_name="c", subcore_axis_name="s")

    @functools.partial(
        pl.kernel,
        mesh=mesh,
        out_type=jax.ShapeDtypeStruct((n_tokens,), jnp.int32),
        scratch_types=[
            pltpu.VMEM((per_w,), jnp.int32),
            pltpu.VMEM((per_w,), jnp.int32),
            pltpu.SemaphoreType.DMA,
        ],
    )
    def router(ids_hbm, table_hbm, out_hbm, idx_v, vals_v, sem):
        wid = lax.axis_index("s") * _NC + lax.axis_index("c")
        base = wid * per_w
        pltpu.sync_copy(ids_hbm.at[pl.ds(base, per_w)], idx_v)
        pltpu.async_copy(table_hbm.at[idx_v], vals_v, sem).wait()
        pltpu.sync_copy(vals_v, out_hbm.at[pl.ds(base, per_w)])

    return router


def kernel(input, hash):
    b, s = input.shape
    n = b * s
    ids = input.astype(jnp.int32).reshape(n)
    out = _make_router(n)(ids, hash.astype(jnp.int32))
    return out.reshape(b, s).astype(hash.dtype)


# speedup vs baseline: 1.0323x; 1.0008x over previous
import functools

import jax
import jax.numpy as jnp
from jax import lax
from jax.experimental import pallas as pl
from jax.experimental.pallas import tpu as pltpu
from jax.experimental.pallas import tpu_sc as plsc

_info = plsc.get_sparse_core_info()
_NC, _NS = _info.num_cores, _info.num_subcores
_NW = _NC * _NS


def _make_router(n_tokens):
    assert n_tokens % (16 * _NW) == 0
    per_w = n_tokens // _NW
    half = per_w // 2
    mesh = plsc.VectorSubcoreMesh(core_axis_name="c", subcore_axis_name="s")

    @functools.partial(
        pl.kernel,
        mesh=mesh,
        out_type=jax.ShapeDtypeStruct((n_tokens,), jnp.int32),
        scratch_types=[
            pltpu.VMEM((half,), jnp.int32),
            pltpu.VMEM((half,), jnp.int32),
            pltpu.VMEM((half,), jnp.int32),
            pltpu.VMEM((half,), jnp.int32),
            pltpu.SemaphoreType.DMA,
            pltpu.SemaphoreType.DMA,
            pltpu.SemaphoreType.DMA,
        ],
    )
    def router(
        ids_hbm, table_hbm, out_hbm, idx0, idx1, val0, val1, s_in, s_g, s_o
    ):
        wid = lax.axis_index("s") * _NC + lax.axis_index("c")
        base = wid * per_w
        in0 = pltpu.async_copy(ids_hbm.at[pl.ds(base, half)], idx0, s_in)
        in1 = pltpu.async_copy(ids_hbm.at[pl.ds(base + half, half)], idx1, s_in)
        in0.wait()
        g0 = pltpu.async_copy(table_hbm.at[idx0], val0, s_g)
        in1.wait()
        g1 = pltpu.async_copy(table_hbm.at[idx1], val1, s_g)
        g0.wait()
        o0 = pltpu.async_copy(val0, out_hbm.at[pl.ds(base, half)], s_o)
        g1.wait()
        o1 = pltpu.async_copy(val1, out_hbm.at[pl.ds(base + half, half)], s_o)
        o0.wait()
        o1.wait()

    return router


def kernel(input, hash):
    b, s = input.shape
    n = b * s
    ids = input.astype(jnp.int32).reshape(n)
    out = _make_router(n)(ids, hash.astype(jnp.int32))
    return out.reshape(b, s).astype(hash.dtype)


# speedup vs baseline: 1.0729x; 1.0394x over previous
import functools

import jax
import jax.numpy as jnp
from jax import lax
from jax.experimental import pallas as pl
from jax.experimental.pallas import tpu as pltpu
from jax.experimental.pallas import tpu_sc as plsc

_info = plsc.get_sparse_core_info()
_NC, _NS = _info.num_cores, _info.num_subcores
_NW = _NC * _NS


def _make_router(n_tokens, vocab):
    assert n_tokens % (8 * _NW) == 0
    per_w = n_tokens // _NW
    mesh = plsc.VectorSubcoreMesh(core_axis_name="c", subcore_axis_name="s")

    @functools.partial(
        pl.kernel,
        mesh=mesh,
        out_type=jax.ShapeDtypeStruct((n_tokens,), jnp.int32),
        scratch_types=[
            pltpu.VMEM((per_w,), jnp.int32),
            pltpu.VMEM((per_w,), jnp.int32),
            pltpu.VMEM_SHARED((vocab,), jnp.int32),
            pltpu.SemaphoreType.DMA,
            pltpu.SemaphoreType.DMA,
        ],
    )
    def router(ids_hbm, table_hbm, out_hbm, idx_v, vals_v, table_s, s_in, s_g):
        sid = lax.axis_index("s")
        wid = sid * _NC + lax.axis_index("c")
        base = wid * per_w
        in_c = pltpu.async_copy(ids_hbm.at[pl.ds(base, per_w)], idx_v, s_in)

        @pl.when(sid == 0)
        def _stage_table():
            pltpu.sync_copy(table_hbm, table_s)

        plsc.subcore_barrier()
        in_c.wait()
        pltpu.async_copy(table_s.at[idx_v], vals_v, s_g).wait()
        pltpu.sync_copy(vals_v, out_hbm.at[pl.ds(base, per_w)])

    return router


def kernel(input, hash):
    b, s = input.shape
    n = b * s
    ids = input.astype(jnp.int32).reshape(n)
    out = _make_router(n, hash.shape[0])(ids, hash.astype(jnp.int32))
    return out.reshape(b, s).astype(hash.dtype)
